# Initial kernel scaffold; baseline (speedup 1.0000x reference)
import jax
import jax.numpy as jnp
from jax.experimental import pallas as pl


def kernel(y_preds, y):
    raise NotImplementedError("write your pallas kernel here")


# speedup vs baseline: 23.7770x; 23.7770x over previous
import functools

import jax
import jax.numpy as jnp
from jax.experimental import pallas as pl
from jax.experimental.pallas import tpu as pltpu


def _wce_body(x_ref, yl_ref, out_ref):
    s = pl.program_id(1)
    W = yl_ref.shape[1]
    yl = yl_ref[...]
    ci = jax.lax.rem(jax.lax.broadcasted_iota(jnp.int32, (1, W), 1), 5)
    base = ci == 0
    gy = jnp.where(yl == 0, 2, jnp.where(yl <= 2, 1, 0))
    total = None
    for t in range(x_ref.shape[0]):
        xt = x_ref[t]
        rm = jnp.max(xt, axis=1, keepdims=True)
        e = jnp.exp(xt - rm)
        r1 = pltpu.roll(xt, W - 1, 1)
        r2 = pltpu.roll(xt, W - 2, 1)
        r3 = pltpu.roll(xt, W - 3, 1)
        r4 = pltpu.roll(xt, W - 4, 1)
        e1 = pltpu.roll(e, W - 1, 1)
        e2 = pltpu.roll(e, W - 2, 1)
        e3 = pltpu.roll(e, W - 3, 1)
        e4 = pltpu.roll(e, W - 4, 1)
        ssum = (e + e1) + (e2 + e3) + e4
        lse = jnp.log(ssum) + rm
        m1 = jnp.maximum(r1, r2)
        m0 = jnp.maximum(r3, r4)
        m = jnp.maximum(xt, jnp.maximum(m1, m0))
        xl = jnp.where(yl == 1, r1, xt)
        xl = jnp.where(yl == 2, r2, xl)
        xl = jnp.where(yl == 3, r3, xl)
        xl = jnp.where(yl == 4, r4, xl)
        gpred = jnp.where(xt == m, 2, jnp.where(m1 == m, 1, 0))
        w = jnp.where(gpred != gy, 1.5, 1.0)
        wce = w * (lse - xl)
        contrib = jnp.where(base, wce, 0.0)
        total = contrib if total is None else total + contrib
    psum = jnp.sum(total, axis=0, keepdims=True)

    @pl.when(s == 0)
    def _init():
        out_ref[...] = jnp.zeros_like(out_ref)

    out_ref[0] += psum


def kernel(y_preds, y):
    T, B, C = y_preds.shape
    G = B // 128
    W = C * 128
    xp = y_preds.reshape(T, G, W)
    yl = jnp.repeat(y.astype(jnp.int32), C).reshape(G, W)
    P, S = 8, 4
    R = G // (P * S)
    out = pl.pallas_call(
        _wce_body,
        grid=(P, S),
        in_specs=[
            pl.BlockSpec((T, R, W), functools.partial(
                lambda S_, p, s: (0, p * S_ + s, 0), S)),
            pl.BlockSpec((R, W), functools.partial(
                lambda S_, p, s: (p * S_ + s, 0), S)),
        ],
        out_specs=pl.BlockSpec((1, 1, W), lambda p, s: (p, 0, 0)),
        out_shape=jax.ShapeDtypeStruct((P, 1, W), jnp.float32),
        compiler_params=pltpu.CompilerParams(
            dimension_semantics=("parallel", "arbitrary")),
    )(xp, yl)
    return (jnp.sum(out) / B).reshape(1)
